# baseline (speedup 1.0000x reference)
import functools

import jax
import jax.numpy as jnp
from jax import lax
from jax.experimental import pallas as pl
from jax.experimental.pallas import tpu as pltpu
from jax.experimental.pallas import tpu_sc as plsc

T = 200
B = 4096
V = 100000
D = 64
DP = 128
DO = D + 2

NC = 2
NS = 16
NW = NC * NS

ROWS = T * B
ROWS_PER_W = ROWS // NW
SUB = 128
CHUNK = 128
CHUNKS = ROWS_PER_W // CHUNK

_mesh = plsc.VectorSubcoreMesh(
    core_axis_name="c", subcore_axis_name="s", num_cores=NC, num_subcores=NS
)


def _transpose_kernel(x_ref, y_ref):
    y_ref[...] = jnp.transpose(x_ref[...], (2, 0, 1))


_TR_T = 8
_TR_B = 512


@jax.jit
def _to_planes(x):
    return pl.pallas_call(
        _transpose_kernel,
        grid=(T // _TR_T, B // _TR_B),
        in_specs=[pl.BlockSpec((_TR_T, _TR_B, DO), lambda i, j: (i, j, 0))],
        out_specs=pl.BlockSpec((DO, _TR_T, _TR_B), lambda i, j: (0, i, j)),
        out_shape=jax.ShapeDtypeStruct((DO, T, B), jnp.float32),
    )(x)


def _pad_table_kernel(table_ref, out_ref):
    out_ref[:, :D] = table_ref[...]


_PAD_ROWS = 1000


@jax.jit
def _pad_table(table):
    return pl.pallas_call(
        _pad_table_kernel,
        grid=(V // _PAD_ROWS,),
        in_specs=[pl.BlockSpec((_PAD_ROWS, D), lambda i: (i, 0))],
        out_specs=pl.BlockSpec((_PAD_ROWS, DP), lambda i: (i, 0)),
        out_shape=jax.ShapeDtypeStruct((V, DP), jnp.float32),
    )(table)


@functools.partial(
    pl.kernel,
    out_type=jax.ShapeDtypeStruct((ROWS, DO), jnp.float32),
    mesh=_mesh,
    scratch_types=[
        pltpu.VMEM((CHUNKS, SUB), jnp.int32),
        pltpu.VMEM((3, CHUNK, DP), jnp.float32),
        pltpu.VMEM((3, CHUNK, DO), jnp.float32),
        pltpu.VMEM((512,), jnp.float32),
        pltpu.SemaphoreType.DMA,
        pltpu.SemaphoreType.DMA,
        pltpu.SemaphoreType.DMA,
        pltpu.SemaphoreType.DMA,
        pltpu.SemaphoreType.DMA,
        pltpu.SemaphoreType.DMA,
    ],
)
def _encode(
    idx_hbm, table_hbm, tf_hbm, out_hbm,
    idx_v, emb_v, stage_v, tf_v,
    gsem0, gsem1, gsem2, wsem0, wsem1, wsem2,
):
    wid = lax.axis_index("s") * NC + lax.axis_index("c")
    pltpu.sync_copy(tf_hbm, tf_v)
    pltpu.sync_copy(idx_hbm.at[pl.ds(wid * CHUNKS, CHUNKS)], idx_v)

    lane = lax.iota(jnp.int32, 16)
    row_base0 = wid * ROWS_PER_W

    gsems = (gsem0, gsem1, gsem2)
    wsems = (wsem0, wsem1, wsem2)

    def start_gather(c, b):
        pltpu.async_copy(table_hbm.at[idx_v.at[c]], emb_v.at[b], gsems[b])

    def process_chunk(c, b):
        base = row_base0 + c * CHUNK
        pltpu.make_async_copy(
            table_hbm.at[idx_v.at[c]], emb_v.at[b], gsems[b]
        ).wait()

        @pl.when(c < CHUNKS - 2)
        def prefetch():
            start_gather(c + 2, (b + 2) % 3)

        @pl.when(c >= 3)
        def wait_stage():
            pltpu.make_async_copy(
                stage_v.at[b],
                out_hbm.at[pl.ds(row_base0 + (c - 3) * CHUNK, CHUNK)],
                wsems[b],
            ).wait()

        @pl.when(jnp.logical_or(c < 3, (base & (B - 1)) < 3 * CHUNK))
        def fill_time():
            t = jnp.right_shift(base, 12)
            tv = tf_v[pl.ds(2 * t, 16)]
            pat = jnp.where(lane == 14, tv[0], tv[1])

            @pl.loop(0, CHUNK, unroll=8)
            def fill(r):
                stage_v[b, r, pl.ds(DO - 16, 16)] = pat

        @pl.loop(0, CHUNK, unroll=8)
        def assemble(r):
            for k in range(D // 16):
                stage_v[b, r, pl.ds(k * 16, 16)] = emb_v[b, r, pl.ds(k * 16, 16)]

        pltpu.async_copy(
            stage_v.at[b], out_hbm.at[pl.ds(base, CHUNK)], wsems[b]
        )

    start_gather(0, 0)
    start_gather(1, 1)

    @pl.loop(0, CHUNKS)
    def chunk_loop(c):
        m = c % 3

        @pl.when(m == 0)
        def p0():
            process_chunk(c, 0)

        @pl.when(m == 1)
        def p1():
            process_chunk(c, 1)

        @pl.when(m == 2)
        def p2():
            process_chunk(c, 2)

    for bb in range(3):
        cc = CHUNKS - 3 + bb
        pltpu.make_async_copy(
            stage_v.at[cc % 3],
            out_hbm.at[pl.ds(row_base0 + cc * CHUNK, CHUNK)],
            wsems[cc % 3],
        ).wait()


def kernel(input, table):
    idx = input[:, :, 0].astype(jnp.int32).reshape(ROWS // SUB, SUB)
    tablep = _pad_table(table)
    t = jnp.arange(T, dtype=jnp.float32)
    tf = jnp.stack([jnp.log(t + 1.0), jnp.exp(t / 1000.0) - 1.0], axis=-1)
    tf = jnp.pad(tf.reshape(-1), (0, 512 - 2 * T))
    out = _encode(idx, tablep, tf)
    planes = _to_planes(out.reshape(T, B, DO))
    return jnp.transpose(planes, (1, 2, 0))


# speedup vs baseline: 1.0698x; 1.0698x over previous
import functools

import jax
import jax.numpy as jnp
from jax import lax
from jax.experimental import pallas as pl
from jax.experimental.pallas import tpu as pltpu
from jax.experimental.pallas import tpu_sc as plsc

T = 200
B = 4096
V = 100000
D = 64
DP = 128
DO = D + 2

NC = 2
NS = 16
NW = NC * NS

ROWS = T * B
ROWS_PER_W = ROWS // NW
SUB = 128
CHUNK = 128
CHUNKS = ROWS_PER_W // CHUNK

_mesh = plsc.VectorSubcoreMesh(
    core_axis_name="c", subcore_axis_name="s", num_cores=NC, num_subcores=NS
)


def _pad_table_kernel(table_ref, out_ref):
    out_ref[:, :D] = table_ref[...]


_PAD_ROWS = 1000


@jax.jit
def _pad_table(table):
    return pl.pallas_call(
        _pad_table_kernel,
        grid=(V // _PAD_ROWS,),
        in_specs=[pl.BlockSpec((_PAD_ROWS, D), lambda i: (i, 0))],
        out_specs=pl.BlockSpec((_PAD_ROWS, DP), lambda i: (i, 0)),
        out_shape=jax.ShapeDtypeStruct((V, DP), jnp.float32),
    )(table)


@functools.partial(
    pl.kernel,
    out_type=jax.ShapeDtypeStruct((ROWS, DO), jnp.float32),
    mesh=_mesh,
    scratch_types=[
        pltpu.VMEM((CHUNKS, SUB), jnp.int32),
        pltpu.VMEM((3, CHUNK, DP), jnp.float32),
        pltpu.VMEM((3, CHUNK, DO), jnp.float32),
        pltpu.VMEM((512,), jnp.float32),
        pltpu.SemaphoreType.DMA,
        pltpu.SemaphoreType.DMA,
        pltpu.SemaphoreType.DMA,
        pltpu.SemaphoreType.DMA,
        pltpu.SemaphoreType.DMA,
        pltpu.SemaphoreType.DMA,
    ],
)
def _encode(
    idx_hbm, table_hbm, tf_hbm, out_hbm,
    idx_v, emb_v, stage_v, tf_v,
    gsem0, gsem1, gsem2, wsem0, wsem1, wsem2,
):
    wid = lax.axis_index("s") * NC + lax.axis_index("c")
    pltpu.sync_copy(tf_hbm, tf_v)
    pltpu.sync_copy(idx_hbm.at[pl.ds(wid * CHUNKS, CHUNKS)], idx_v)

    lane = lax.iota(jnp.int32, 16)
    row_base0 = wid * ROWS_PER_W

    gsems = (gsem0, gsem1, gsem2)
    wsems = (wsem0, wsem1, wsem2)

    def start_gather(c, b):
        pltpu.async_copy(table_hbm.at[idx_v.at[c]], emb_v.at[b], gsems[b])

    def process_chunk(c, b):
        base = row_base0 + c * CHUNK
        pltpu.make_async_copy(
            table_hbm.at[idx_v.at[c]], emb_v.at[b], gsems[b]
        ).wait()

        @pl.when(c < CHUNKS - 2)
        def prefetch():
            start_gather(c + 2, (b + 2) % 3)

        @pl.when(c >= 3)
        def wait_stage():
            pltpu.make_async_copy(
                stage_v.at[b],
                out_hbm.at[pl.ds(row_base0 + (c - 3) * CHUNK, CHUNK)],
                wsems[b],
            ).wait()

        @pl.when(jnp.logical_or(c < 3, (base & (B - 1)) < 3 * CHUNK))
        def fill_time():
            t = jnp.right_shift(base, 12)
            tv = tf_v[pl.ds(2 * t, 16)]
            pat = jnp.where(lane == 14, tv[0], tv[1])

            @pl.loop(0, CHUNK, unroll=8)
            def fill(r):
                stage_v[b, r, pl.ds(DO - 16, 16)] = pat

        @pl.loop(0, CHUNK, unroll=8)
        def assemble(r):
            for k in range(D // 16):
                stage_v[b, r, pl.ds(k * 16, 16)] = emb_v[b, r, pl.ds(k * 16, 16)]

        pltpu.async_copy(
            stage_v.at[b], out_hbm.at[pl.ds(base, CHUNK)], wsems[b]
        )

    start_gather(0, 0)
    start_gather(1, 1)

    @pl.loop(0, CHUNKS)
    def chunk_loop(c):
        m = c % 3

        @pl.when(m == 0)
        def p0():
            process_chunk(c, 0)

        @pl.when(m == 1)
        def p1():
            process_chunk(c, 1)

        @pl.when(m == 2)
        def p2():
            process_chunk(c, 2)

    for bb in range(3):
        cc = CHUNKS - 3 + bb
        pltpu.make_async_copy(
            stage_v.at[cc % 3],
            out_hbm.at[pl.ds(row_base0 + cc * CHUNK, CHUNK)],
            wsems[cc % 3],
        ).wait()


def kernel(input, table):
    idx = input[:, :, 0].astype(jnp.int32).reshape(ROWS // SUB, SUB)
    tablep = _pad_table(table)
    t = jnp.arange(T, dtype=jnp.float32)
    tf = jnp.stack([jnp.log(t + 1.0), jnp.exp(t / 1000.0) - 1.0], axis=-1)
    tf = jnp.pad(tf.reshape(-1), (0, 512 - 2 * T))
    out = _encode(idx, tablep, tf)
    return out.reshape(T, B, DO)
